# baseline (speedup 1.0000x reference)
import dataclasses

import jax
import jax.numpy as jnp
from jax import lax
from jax.experimental import pallas as pl
from jax.experimental.pallas import tpu as pltpu
from jax.experimental.pallas import tpu_sc as plsc

_VOCAB = 100000
_DIM = 128
_B = 16384
_NEG = 20
_K = _NEG + 1
_NC = 2
_NS = 16
_NW = _NC * _NS
_BW = _B // _NW
_G = 16
_NCHUNK = _BW // _G
_DOTS = _G * _K
_L = 16
_L2 = 32
_NCH = _DIM // _L
_NCH2 = _DIM // _L2
_GRPS = ((0, 128), (128, 128), (256, 80))


def _sc_scores_body(v_hbm, u_hbm, cidx_hbm, uidx_hbm, scores_hbm,
                    cidx0_v, cidx1_v, uidx0_v, uidx1_v, vc0_v, vc1_v,
                    u0_v, u1_v, acc_v, sc0_v, sc1_v,
                    semv0, semv1, semu0, semu1, sems0, sems1):
    wid = lax.axis_index("s") * _NC + lax.axis_index("c")
    cidx = (cidx0_v, cidx1_v)
    uidx = (uidx0_v, uidx1_v)
    vc = (vc0_v, vc1_v)
    uu = (u0_v, u1_v)
    scv = (sc0_v, sc1_v)
    semv = (semv0, semv1)
    semu = (semu0, semu1)
    sems = (sems0, sems1)

    def stage(b, j):
        cb = wid * _BW + j * _G
        pltpu.sync_copy(cidx_hbm.at[pl.ds(cb, _G)], cidx[b])
        pltpu.sync_copy(uidx_hbm.at[pl.ds(cb * _K, _DOTS)], uidx[b])

    def fire(b):
        pltpu.async_copy(v_hbm.at[cidx[b]], vc[b], semv[b])
        for off, sz in _GRPS:
            pltpu.async_copy(
                u_hbm.at[uidx[b].at[pl.ds(off, sz)]],
                uu[b].at[pl.ds(off, sz)], semu[b])

    def drain(b):
        pltpu.make_async_copy(v_hbm.at[cidx[b]], vc[b], semv[b]).wait()
        for off, sz in _GRPS:
            pltpu.make_async_copy(
                u_hbm.at[uidx[b].at[pl.ds(off, sz)]],
                uu[b].at[pl.ds(off, sz)], semu[b]).wait()

    def drain_scores(b, j):
        pltpu.make_async_copy(
            scv[b],
            scores_hbm.at[pl.ds((wid * _BW + j * _G) * _K, _DOTS)],
            sems[b]).wait()

    def compute(b, j):
        cb = wid * _BW + j * _G
        vc_b, u_b, sc_b = vc[b], uu[b], scv[b]

        @pl.loop(0, _G)
        def _per_b(g):
            vcc = [plsc.bitcast(vc_b[g, pl.ds(c * _L, _L)], jnp.bfloat16)
                   for c in range(_NCH2)]
            for k in range(_K):
                r = g * _K + k
                acc = vcc[0] * plsc.bitcast(
                    u_b[r, pl.ds(0, _L)], jnp.bfloat16)
                for c in range(1, _NCH2):
                    uc = plsc.bitcast(
                        u_b[r, pl.ds(c * _L, _L)], jnp.bfloat16)
                    acc = acc + vcc[c] * uc
                ev, od = plsc.unpack(acc, format=plsc.PackFormat.INTERLEAVED)
                acc_v[pl.ds(r * _L, _L)] = ev + od

        @pl.when(j >= 2)
        def _():
            drain_scores(b, j - 2)

        @pl.loop(0, _K)
        def _reduce(t):
            tot = None
            for l in range(_L):
                gi = lax.iota(jnp.int32, _L) * _L + (t * (_L * _L) + l)
                gl = plsc.load_gather(acc_v, [gi])
                tot = gl if tot is None else tot + gl
            sc_b[pl.ds(t * _L, _L)] = tot

        pltpu.async_copy(sc_b, scores_hbm.at[pl.ds(cb * _K, _DOTS)],
                         sems[b])

    stage(0, 0)
    fire(0)

    @pl.loop(0, _NCHUNK, step=2)
    def _chunks(j):
        stage(1, j + 1)
        fire(1)
        drain(0)
        compute(0, j)

        @pl.when(j + 2 < _NCHUNK)
        def _():
            stage(0, j + 2)
            fire(0)

        drain(1)
        compute(1, j + 1)

    drain_scores(0, _NCHUNK - 2)
    drain_scores(1, _NCHUNK - 1)


def _tc_loss_body(s_ref, o_ref):
    s = s_ref[...]
    n_rows = _B * _K // 128
    rows = lax.broadcasted_iota(jnp.int32, (n_rows, 128), 0)
    cols = lax.broadcasted_iota(jnp.int32, (n_rows, 128), 1)
    didx = rows * 128 + cols
    x = jnp.where(didx % _K == 0, s, -s)
    term = jnp.log(jax.nn.sigmoid(x) + 1e-10)
    o_ref[0, 0] = -jnp.sum(term) / _B


def kernel(center, context, negatives, V, U):
    center = center.astype(jnp.int32)
    uidx = jnp.concatenate([context[:, None], negatives], axis=1)
    uidx = uidx.reshape(_B * _K).astype(jnp.int32)
    def _pack(x):
        w = jax.lax.bitcast_convert_type(
            x.astype(jnp.bfloat16), jnp.uint16).astype(jnp.uint32)
        word = w[:, :_DIM // 2] | (w[:, _DIM // 2:] << 16)
        return jax.lax.bitcast_convert_type(word, jnp.int32)

    vtab = _pack(V)
    utab = _pack(U)

    cp = pltpu.CompilerParams()
    if "needs_layout_passes" in pltpu.CompilerParams.__dataclass_fields__:
        cp = dataclasses.replace(cp, needs_layout_passes=False)
    if "use_tc_tiling_on_sc" in pltpu.CompilerParams.__dataclass_fields__:
        cp = dataclasses.replace(cp, use_tc_tiling_on_sc=False)
    mesh = plsc.VectorSubcoreMesh(core_axis_name="c", subcore_axis_name="s")
    sc = pl.kernel(
        _sc_scores_body,
        out_type=jax.ShapeDtypeStruct((_B * _K,), jnp.float32),
        mesh=mesh,
        compiler_params=cp,
        scratch_types=[
            pltpu.VMEM((_G,), jnp.int32),
            pltpu.VMEM((_G,), jnp.int32),
            pltpu.VMEM((_DOTS,), jnp.int32),
            pltpu.VMEM((_DOTS,), jnp.int32),
            pltpu.VMEM((_G, _DIM // 2), jnp.int32),
            pltpu.VMEM((_G, _DIM // 2), jnp.int32),
            pltpu.VMEM((_DOTS, _DIM // 2), jnp.int32),
            pltpu.VMEM((_DOTS, _DIM // 2), jnp.int32),
            pltpu.VMEM((_DOTS * _L,), jnp.float32),
            pltpu.VMEM((_DOTS,), jnp.float32),
            pltpu.VMEM((_DOTS,), jnp.float32),
            pltpu.SemaphoreType.DMA,
            pltpu.SemaphoreType.DMA,
            pltpu.SemaphoreType.DMA,
            pltpu.SemaphoreType.DMA,
            pltpu.SemaphoreType.DMA,
            pltpu.SemaphoreType.DMA,
        ],
    )
    scores = sc(vtab, utab, center, uidx)

    s2 = scores.reshape(_B * _K // 128, 128)
    out = pl.pallas_call(
        _tc_loss_body,
        out_shape=jax.ShapeDtypeStruct((1, 1), jnp.float32),
        out_specs=pl.BlockSpec(memory_space=pltpu.SMEM),
    )(s2)
    return out[0, 0]


# speedup vs baseline: 1.1552x; 1.1552x over previous
import dataclasses

import jax
import jax.numpy as jnp
from jax import lax
from jax.experimental import pallas as pl
from jax.experimental.pallas import tpu as pltpu
from jax.experimental.pallas import tpu_sc as plsc

_VOCAB = 100000
_DIM = 128
_B = 16384
_NEG = 20
_K = _NEG + 1
_NC = 2
_NS = 16
_NW = _NC * _NS
_BW = _B // _NW
_G = 16
_NCHUNK = _BW // _G
_DOTS = _G * _K
_L = 16
_L2 = 32
_NCH = _DIM // _L
_NCH2 = _DIM // _L2
_GRPS = ((0, 128), (128, 128), (256, 80))


def _sc_scores_body(v_hbm, u_hbm, cidx_hbm, uidx_hbm, scores_hbm,
                    cidx0_v, cidx1_v, uidx0_v, uidx1_v, vc0_v, vc1_v,
                    u0_v, u1_v, acc_v, sc0_v, sc1_v,
                    semv0, semv1, semu0, semu1, sems0, sems1):
    wid = lax.axis_index("s") * _NC + lax.axis_index("c")
    cidx = (cidx0_v, cidx1_v)
    uidx = (uidx0_v, uidx1_v)
    vc = (vc0_v, vc1_v)
    uu = (u0_v, u1_v)
    scv = (sc0_v, sc1_v)
    semv = (semv0, semv1)
    semu = (semu0, semu1)
    sems = (sems0, sems1)

    def stage(b, j):
        cb = wid * _BW + j * _G
        pltpu.sync_copy(cidx_hbm.at[pl.ds(cb, _G)], cidx[b])
        pltpu.sync_copy(uidx_hbm.at[pl.ds(cb * _K, _DOTS)], uidx[b])

    def fire(b):
        pltpu.async_copy(v_hbm.at[cidx[b]], vc[b], semv[b])
        for off, sz in _GRPS:
            pltpu.async_copy(
                u_hbm.at[uidx[b].at[pl.ds(off, sz)]],
                uu[b].at[pl.ds(off, sz)], semu[b])

    def drain(b):
        pltpu.make_async_copy(v_hbm.at[cidx[b]], vc[b], semv[b]).wait()
        for off, sz in _GRPS:
            pltpu.make_async_copy(
                u_hbm.at[uidx[b].at[pl.ds(off, sz)]],
                uu[b].at[pl.ds(off, sz)], semu[b]).wait()

    def drain_scores(b, j):
        pltpu.make_async_copy(
            scv[b],
            scores_hbm.at[pl.ds((wid * _BW + j * _G) * _K, _DOTS)],
            sems[b]).wait()

    def compute(b, j):
        cb = wid * _BW + j * _G
        vc_b, u_b, sc_b = vc[b], uu[b], scv[b]

        @plsc.parallel_loop(0, _G, 1, unroll=2)
        def _per_b(g):
            vcc = [plsc.bitcast(vc_b[g, pl.ds(c * _L, _L)], jnp.bfloat16)
                   for c in range(_NCH2)]
            for k in range(_K):
                r = g * _K + k
                acc = vcc[0] * plsc.bitcast(
                    u_b[r, pl.ds(0, _L)], jnp.bfloat16)
                for c in range(1, _NCH2):
                    uc = plsc.bitcast(
                        u_b[r, pl.ds(c * _L, _L)], jnp.bfloat16)
                    acc = acc + vcc[c] * uc
                ev, od = plsc.unpack(acc, format=plsc.PackFormat.INTERLEAVED)
                acc_v[pl.ds(r * _L, _L)] = ev + od

        @pl.when(j >= 2)
        def _():
            drain_scores(b, j - 2)

        @plsc.parallel_loop(0, _K, 1, unroll=2)
        def _reduce(t):
            tot = None
            for l in range(_L):
                gi = lax.iota(jnp.int32, _L) * _L + (t * (_L * _L) + l)
                gl = plsc.load_gather(acc_v, [gi])
                tot = gl if tot is None else tot + gl
            sc_b[pl.ds(t * _L, _L)] = tot

        pltpu.async_copy(sc_b, scores_hbm.at[pl.ds(cb * _K, _DOTS)],
                         sems[b])

    stage(0, 0)
    fire(0)

    @pl.loop(0, _NCHUNK, step=2)
    def _chunks(j):
        stage(1, j + 1)
        fire(1)
        drain(0)
        compute(0, j)

        @pl.when(j + 2 < _NCHUNK)
        def _():
            stage(0, j + 2)
            fire(0)

        drain(1)
        compute(1, j + 1)

    drain_scores(0, _NCHUNK - 2)
    drain_scores(1, _NCHUNK - 1)


def _tc_loss_body(s_ref, o_ref):
    s = s_ref[...]
    n_rows = _B * _K // 128
    rows = lax.broadcasted_iota(jnp.int32, (n_rows, 128), 0)
    cols = lax.broadcasted_iota(jnp.int32, (n_rows, 128), 1)
    didx = rows * 128 + cols
    x = jnp.where(didx % _K == 0, s, -s)
    term = jnp.log(jax.nn.sigmoid(x) + 1e-10)
    o_ref[0, 0] = -jnp.sum(term) / _B


def kernel(center, context, negatives, V, U):
    center = center.astype(jnp.int32)
    uidx = jnp.concatenate([context[:, None], negatives], axis=1)
    uidx = uidx.reshape(_B * _K).astype(jnp.int32)
    def _pack(x):
        w = jax.lax.bitcast_convert_type(
            x.astype(jnp.bfloat16), jnp.uint16).astype(jnp.uint32)
        word = w[:, :_DIM // 2] | (w[:, _DIM // 2:] << 16)
        return jax.lax.bitcast_convert_type(word, jnp.int32)

    vtab = _pack(V)
    utab = _pack(U)

    cp = pltpu.CompilerParams()
    if "needs_layout_passes" in pltpu.CompilerParams.__dataclass_fields__:
        cp = dataclasses.replace(cp, needs_layout_passes=False)
    if "use_tc_tiling_on_sc" in pltpu.CompilerParams.__dataclass_fields__:
        cp = dataclasses.replace(cp, use_tc_tiling_on_sc=False)
    mesh = plsc.VectorSubcoreMesh(core_axis_name="c", subcore_axis_name="s")
    sc = pl.kernel(
        _sc_scores_body,
        out_type=jax.ShapeDtypeStruct((_B * _K,), jnp.float32),
        mesh=mesh,
        compiler_params=cp,
        scratch_types=[
            pltpu.VMEM((_G,), jnp.int32),
            pltpu.VMEM((_G,), jnp.int32),
            pltpu.VMEM((_DOTS,), jnp.int32),
            pltpu.VMEM((_DOTS,), jnp.int32),
            pltpu.VMEM((_G, _DIM // 2), jnp.int32),
            pltpu.VMEM((_G, _DIM // 2), jnp.int32),
            pltpu.VMEM((_DOTS, _DIM // 2), jnp.int32),
            pltpu.VMEM((_DOTS, _DIM // 2), jnp.int32),
            pltpu.VMEM((_DOTS * _L,), jnp.float32),
            pltpu.VMEM((_DOTS,), jnp.float32),
            pltpu.VMEM((_DOTS,), jnp.float32),
            pltpu.SemaphoreType.DMA,
            pltpu.SemaphoreType.DMA,
            pltpu.SemaphoreType.DMA,
            pltpu.SemaphoreType.DMA,
            pltpu.SemaphoreType.DMA,
            pltpu.SemaphoreType.DMA,
        ],
    )
    scores = sc(vtab, utab, center, uidx)

    s2 = scores.reshape(_B * _K // 128, 128)
    out = pl.pallas_call(
        _tc_loss_body,
        out_shape=jax.ShapeDtypeStruct((1, 1), jnp.float32),
        out_specs=pl.BlockSpec(memory_space=pltpu.SMEM),
    )(s2)
    return out[0, 0]


# speedup vs baseline: 1.5124x; 1.3092x over previous
import dataclasses

import jax
import jax.numpy as jnp
from jax import lax
from jax.experimental import pallas as pl
from jax.experimental.pallas import tpu as pltpu
from jax.experimental.pallas import tpu_sc as plsc

_VOCAB = 100000
_DIM = 128
_B = 16384
_NEG = 20
_K = _NEG + 1
_NC = 2
_NS = 16
_NW = _NC * _NS
_BW = _B // _NW
_G = 16
_NCHUNK = _BW // _G
_DOTS = _G * _K
_L = 16
_L2 = 32
_NCH = _DIM // _L
_NCH2 = _DIM // _L2
_GRPS = ((0, 128), (128, 128), (256, 80))


def _sc_scores_body(v_hbm, u_hbm, cidx_hbm, uidx_hbm, scores_hbm,
                    cidx0_v, cidx1_v, uidx0_v, uidx1_v, vc0_v, vc1_v,
                    u0_v, u1_v, acc_v, sc0_v, sc1_v,
                    semv0, semv1, semu0, semu1, sems0, sems1):
    wid = lax.axis_index("s") * _NC + lax.axis_index("c")
    cidx = (cidx0_v, cidx1_v)
    uidx = (uidx0_v, uidx1_v)
    vc = (vc0_v, vc1_v)
    uu = (u0_v, u1_v)
    scv = (sc0_v, sc1_v)
    semv = (semv0, semv1)
    semu = (semu0, semu1)
    sems = (sems0, sems1)

    def stage(b, j):
        cb = wid * _BW + j * _G
        pltpu.sync_copy(cidx_hbm.at[pl.ds(cb, _G)], cidx[b])
        pltpu.sync_copy(uidx_hbm.at[pl.ds(cb * _K, _DOTS)], uidx[b])

    def fire(b):
        pltpu.async_copy(v_hbm.at[cidx[b]], vc[b], semv[b])
        for off, sz in _GRPS:
            pltpu.async_copy(
                u_hbm.at[uidx[b].at[pl.ds(off, sz)]],
                uu[b].at[pl.ds(off, sz)], semu[b])

    def drain(b):
        pltpu.make_async_copy(v_hbm.at[cidx[b]], vc[b], semv[b]).wait()
        for off, sz in _GRPS:
            pltpu.make_async_copy(
                u_hbm.at[uidx[b].at[pl.ds(off, sz)]],
                uu[b].at[pl.ds(off, sz)], semu[b]).wait()

    def drain_scores(b, j):
        pltpu.make_async_copy(
            scv[b],
            scores_hbm.at[pl.ds((wid * _BW + j * _G) * _K, _DOTS)],
            sems[b]).wait()

    def compute(b, j):
        cb = wid * _BW + j * _G
        vc_b, u_b, sc_b = vc[b], uu[b], scv[b]

        @plsc.parallel_loop(0, _G, 1, unroll=2)
        def _per_b(g):
            vcc = [plsc.pack(vc_b[g, pl.ds(c * _L, _L)],
                             vc_b[g, pl.ds((c + _NCH2) * _L, _L)],
                             format=plsc.PackFormat.INTERLEAVED)
                   for c in range(_NCH2)]
            for k in range(_K):
                r = g * _K + k
                acc = vcc[0] * plsc.bitcast(
                    u_b[r, pl.ds(0, _L)], jnp.bfloat16)
                for c in range(1, _NCH2):
                    uc = plsc.bitcast(
                        u_b[r, pl.ds(c * _L, _L)], jnp.bfloat16)
                    acc = acc + vcc[c] * uc
                ev, od = plsc.unpack(acc, format=plsc.PackFormat.INTERLEAVED)
                acc_v[pl.ds(r * _L, _L)] = ev + od

        @pl.when(j >= 2)
        def _():
            drain_scores(b, j - 2)

        @plsc.parallel_loop(0, _K, 1, unroll=2)
        def _reduce(t):
            tot = None
            for l in range(_L):
                gi = lax.iota(jnp.int32, _L) * _L + (t * (_L * _L) + l)
                gl = plsc.load_gather(acc_v, [gi])
                tot = gl if tot is None else tot + gl
            sc_b[pl.ds(t * _L, _L)] = tot

        pltpu.async_copy(sc_b, scores_hbm.at[pl.ds(cb * _K, _DOTS)],
                         sems[b])

    stage(0, 0)
    fire(0)

    @pl.loop(0, _NCHUNK, step=2)
    def _chunks(j):
        stage(1, j + 1)
        fire(1)
        drain(0)
        compute(0, j)

        @pl.when(j + 2 < _NCHUNK)
        def _():
            stage(0, j + 2)
            fire(0)

        drain(1)
        compute(1, j + 1)

    drain_scores(0, _NCHUNK - 2)
    drain_scores(1, _NCHUNK - 1)


_PROWS = 1000


def _tc_pack_body(u_ref, o_ref):
    u = u_ref[...]
    r = (u + jnp.uint32(0x7FFF) + ((u >> 16) & jnp.uint32(1))) >> 16
    word = r[:, :_DIM // 2] | (r[:, _DIM // 2:] << 16)
    o_ref[...] = jax.lax.bitcast_convert_type(word, jnp.int32)


def _tc_loss_body(s_ref, o_ref):
    s = s_ref[...]
    n_rows = _B * _K // 128
    rows = lax.broadcasted_iota(jnp.int32, (n_rows, 128), 0)
    cols = lax.broadcasted_iota(jnp.int32, (n_rows, 128), 1)
    didx = rows * 128 + cols
    x = jnp.where(didx % _K == 0, s, -s)
    term = jnp.log(jax.nn.sigmoid(x) + 1e-10)
    o_ref[0, 0] = -jnp.sum(term) / _B


def kernel(center, context, negatives, V, U):
    center = center.astype(jnp.int32)
    uidx = jnp.concatenate([context[:, None], negatives], axis=1)
    uidx = uidx.reshape(_B * _K).astype(jnp.int32)
    utab = pl.pallas_call(
        _tc_pack_body,
        grid=(_VOCAB // _PROWS,),
        in_specs=[pl.BlockSpec((_PROWS, _DIM), lambda i: (i, 0))],
        out_specs=pl.BlockSpec((_PROWS, _DIM // 2), lambda i: (i, 0)),
        out_shape=jax.ShapeDtypeStruct((_VOCAB, _DIM // 2), jnp.int32),
    )(jax.lax.bitcast_convert_type(U, jnp.uint32))

    cp = pltpu.CompilerParams()
    if "needs_layout_passes" in pltpu.CompilerParams.__dataclass_fields__:
        cp = dataclasses.replace(cp, needs_layout_passes=False)
    if "use_tc_tiling_on_sc" in pltpu.CompilerParams.__dataclass_fields__:
        cp = dataclasses.replace(cp, use_tc_tiling_on_sc=False)
    mesh = plsc.VectorSubcoreMesh(core_axis_name="c", subcore_axis_name="s")
    sc = pl.kernel(
        _sc_scores_body,
        out_type=jax.ShapeDtypeStruct((_B * _K,), jnp.float32),
        mesh=mesh,
        compiler_params=cp,
        scratch_types=[
            pltpu.VMEM((_G,), jnp.int32),
            pltpu.VMEM((_G,), jnp.int32),
            pltpu.VMEM((_DOTS,), jnp.int32),
            pltpu.VMEM((_DOTS,), jnp.int32),
            pltpu.VMEM((_G, _DIM), jnp.float32),
            pltpu.VMEM((_G, _DIM), jnp.float32),
            pltpu.VMEM((_DOTS, _DIM // 2), jnp.int32),
            pltpu.VMEM((_DOTS, _DIM // 2), jnp.int32),
            pltpu.VMEM((_DOTS * _L,), jnp.float32),
            pltpu.VMEM((_DOTS,), jnp.float32),
            pltpu.VMEM((_DOTS,), jnp.float32),
            pltpu.SemaphoreType.DMA,
            pltpu.SemaphoreType.DMA,
            pltpu.SemaphoreType.DMA,
            pltpu.SemaphoreType.DMA,
            pltpu.SemaphoreType.DMA,
            pltpu.SemaphoreType.DMA,
        ],
    )
    scores = sc(V, utab, center, uidx)

    s2 = scores.reshape(_B * _K // 128, 128)
    out = pl.pallas_call(
        _tc_loss_body,
        out_shape=jax.ShapeDtypeStruct((1, 1), jnp.float32),
        out_specs=pl.BlockSpec(memory_space=pltpu.SMEM),
    )(s2)
    return out[0, 0]


# speedup vs baseline: 2.7019x; 1.7865x over previous
import dataclasses

import jax
import jax.numpy as jnp
from jax import lax
from jax.experimental import pallas as pl
from jax.experimental.pallas import tpu as pltpu
from jax.experimental.pallas import tpu_sc as plsc

_VOCAB = 100000
_DIM = 128
_B = 16384
_NEG = 20
_K = _NEG + 1
_NC = 2
_NS = 16
_NW = _NC * _NS
_BW = _B // _NW
_G = 16
_NCHUNK = _BW // _G
_DOTS = _G * _K
_L = 16
_L2 = 32
_NCH = _DIM // _L
_NCH2 = _DIM // _L2
_GRPS = ((0, 128), (128, 128), (256, 80))


def _sc_scores_body(v_hbm, u_hbm, cidx_hbm, uidx_hbm, scores_hbm,
                    cidx0_v, cidx1_v, uidx0_v, uidx1_v, vc0_v, vc1_v,
                    u0_v, u1_v, acc_v, sc0_v, sc1_v,
                    semv0, semv1, semu0, semu1, sems0, sems1):
    wid = lax.axis_index("s") * _NC + lax.axis_index("c")
    cidx = (cidx0_v, cidx1_v)
    uidx = (uidx0_v, uidx1_v)
    vc = (vc0_v, vc1_v)
    uu = (u0_v, u1_v)
    scv = (sc0_v, sc1_v)
    semv = (semv0, semv1)
    semu = (semu0, semu1)
    sems = (sems0, sems1)

    def stage(b, j):
        cb = wid * _BW + j * _G
        pltpu.sync_copy(cidx_hbm.at[pl.ds(cb, _G)], cidx[b])
        pltpu.sync_copy(uidx_hbm.at[pl.ds(cb * _K, _DOTS)], uidx[b])

    def fire(b):
        pltpu.async_copy(v_hbm.at[cidx[b]], vc[b], semv[b])
        for off, sz in _GRPS:
            pltpu.async_copy(
                u_hbm.at[uidx[b].at[pl.ds(off, sz)]],
                uu[b].at[pl.ds(off, sz)], semu[b])

    def drain(b):
        pltpu.make_async_copy(v_hbm.at[cidx[b]], vc[b], semv[b]).wait()
        for off, sz in _GRPS:
            pltpu.make_async_copy(
                u_hbm.at[uidx[b].at[pl.ds(off, sz)]],
                uu[b].at[pl.ds(off, sz)], semu[b]).wait()

    def drain_scores(b, j):
        pltpu.make_async_copy(
            scv[b],
            scores_hbm.at[pl.ds((wid * _BW + j * _G) * _K, _DOTS)],
            sems[b]).wait()

    def compute(b, j):
        cb = wid * _BW + j * _G
        vc_b, u_b, sc_b = vc[b], uu[b], scv[b]

        @plsc.parallel_loop(0, _G, 1, unroll=2)
        def _per_b(g):
            vcc = [vc_b[g, pl.ds(c * _L, _L)] for c in range(_NCH)]
            for k in range(_K):
                r = g * _K + k
                acc = vcc[0] * u_b[r, pl.ds(0, _L)]
                for c in range(1, _NCH):
                    acc = acc + vcc[c] * u_b[r, pl.ds(c * _L, _L)]
                acc_v[pl.ds(r * _L, _L)] = acc

        @pl.when(j >= 2)
        def _():
            drain_scores(b, j - 2)

        @plsc.parallel_loop(0, _K, 1, unroll=2)
        def _reduce(t):
            tot = None
            for l in range(_L):
                gi = lax.iota(jnp.int32, _L) * _L + (t * (_L * _L) + l)
                gl = plsc.load_gather(acc_v, [gi])
                tot = gl if tot is None else tot + gl
            sc_b[pl.ds(t * _L, _L)] = tot

        pltpu.async_copy(sc_b, scores_hbm.at[pl.ds(cb * _K, _DOTS)],
                         sems[b])

    stage(0, 0)
    fire(0)

    @pl.loop(0, _NCHUNK, step=2)
    def _chunks(j):
        stage(1, j + 1)
        fire(1)
        drain(0)
        compute(0, j)

        @pl.when(j + 2 < _NCHUNK)
        def _():
            stage(0, j + 2)
            fire(0)

        drain(1)
        compute(1, j + 1)

    drain_scores(0, _NCHUNK - 2)
    drain_scores(1, _NCHUNK - 1)


_PROWS = 1000


def _tc_pack_body(u_ref, o_ref):
    u = u_ref[...]
    r = (u + jnp.uint32(0x7FFF) + ((u >> 16) & jnp.uint32(1))) >> 16
    word = r[:, :_DIM // 2] | (r[:, _DIM // 2:] << 16)
    o_ref[...] = jax.lax.bitcast_convert_type(word, jnp.int32)


def _tc_loss_body(s_ref, o_ref):
    s = s_ref[...]
    n_rows = _B * _K // 128
    rows = lax.broadcasted_iota(jnp.int32, (n_rows, 128), 0)
    cols = lax.broadcasted_iota(jnp.int32, (n_rows, 128), 1)
    didx = rows * 128 + cols
    x = jnp.where(didx % _K == 0, s, -s)
    term = jnp.log(jax.nn.sigmoid(x) + 1e-10)
    o_ref[0, 0] = -jnp.sum(term) / _B


def kernel(center, context, negatives, V, U):
    center = center.astype(jnp.int32)
    uidx = jnp.concatenate([context[:, None], negatives], axis=1)
    uidx = uidx.reshape(_B * _K).astype(jnp.int32)

    cp = pltpu.CompilerParams()
    if "needs_layout_passes" in pltpu.CompilerParams.__dataclass_fields__:
        cp = dataclasses.replace(cp, needs_layout_passes=False)
    if "use_tc_tiling_on_sc" in pltpu.CompilerParams.__dataclass_fields__:
        cp = dataclasses.replace(cp, use_tc_tiling_on_sc=False)
    mesh = plsc.VectorSubcoreMesh(core_axis_name="c", subcore_axis_name="s")
    sc = pl.kernel(
        _sc_scores_body,
        out_type=jax.ShapeDtypeStruct((_B * _K,), jnp.float32),
        mesh=mesh,
        compiler_params=cp,
        scratch_types=[
            pltpu.VMEM((_G,), jnp.int32),
            pltpu.VMEM((_G,), jnp.int32),
            pltpu.VMEM((_DOTS,), jnp.int32),
            pltpu.VMEM((_DOTS,), jnp.int32),
            pltpu.VMEM((_G, _DIM), jnp.float32),
            pltpu.VMEM((_G, _DIM), jnp.float32),
            pltpu.VMEM((_DOTS, _DIM), jnp.float32),
            pltpu.VMEM((_DOTS, _DIM), jnp.float32),
            pltpu.VMEM((_DOTS * _L,), jnp.float32),
            pltpu.VMEM((_DOTS,), jnp.float32),
            pltpu.VMEM((_DOTS,), jnp.float32),
            pltpu.SemaphoreType.DMA,
            pltpu.SemaphoreType.DMA,
            pltpu.SemaphoreType.DMA,
            pltpu.SemaphoreType.DMA,
            pltpu.SemaphoreType.DMA,
            pltpu.SemaphoreType.DMA,
        ],
    )
    scores = sc(V, U, center, uidx)

    s2 = scores.reshape(_B * _K // 128, 128)
    out = pl.pallas_call(
        _tc_loss_body,
        out_shape=jax.ShapeDtypeStruct((1, 1), jnp.float32),
        out_specs=pl.BlockSpec(memory_space=pltpu.SMEM),
    )(s2)
    return out[0, 0]
